# baseline (speedup 1.0000x reference)
import functools

import jax
import jax.numpy as jnp
import numpy as np
from jax import lax
from jax.experimental import pallas as pl
from jax.experimental.pallas import tpu as pltpu
from jax.experimental.pallas import tpu_sc as plsc

BATCH = 128
KEEP = 96
D = 512 * 512

NC, NS = 2, 16
NW = NC * NS
ROWS_PER_W = KEEP // NW
CHUNK = 32768
NCHUNKS = D // CHUNK
NTASKS = ROWS_PER_W * NCHUNKS


@functools.partial(
    pl.kernel,
    out_type=(
        jax.ShapeDtypeStruct((KEEP, D), jnp.float32),
        jax.ShapeDtypeStruct((BATCH,), jnp.int32),
    ),
    mesh=plsc.VectorSubcoreMesh(core_axis_name="c", subcore_axis_name="s"),
    scratch_types=[
        pltpu.VMEM((16,), jnp.int32),
        pltpu.VMEM((BATCH,), jnp.int32),
        pltpu.VMEM((CHUNK,), jnp.float32),
        pltpu.VMEM((CHUNK,), jnp.float32),
        pltpu.SemaphoreType.DMA,
        pltpu.SemaphoreType.DMA,
    ],
)
def _sc_gather(x_hbm, idxmat_hbm, perm_hbm, out_hbm, idx_out_hbm,
               idx_v, perm_v, buf0, buf1, sem_in, sem_out):
    wid = lax.axis_index("s") * NC + lax.axis_index("c")

    @pl.when(wid == 0)
    def _():
        pltpu.sync_copy(perm_hbm, perm_v)
        pltpu.sync_copy(perm_v, idx_out_hbm)

    pltpu.sync_copy(idxmat_hbm.at[wid], idx_v)
    vec = idx_v[...]
    srcs = [vec[j] for j in range(ROWS_PER_W)]
    obase = wid * ROWS_PER_W

    bufs = (buf0, buf1)
    tasks = [(j, c) for j in range(ROWS_PER_W) for c in range(NCHUNKS)]

    def start_gather(k, p):
        j, c = tasks[k]
        return pltpu.async_copy(
            x_hbm.at[srcs[j], pl.ds(c * CHUNK, CHUNK)], bufs[p], sem_in
        )

    def start_scatter(k, p):
        j, c = tasks[k]
        return pltpu.async_copy(
            bufs[p], out_hbm.at[obase + j, pl.ds(c * CHUNK, CHUNK)], sem_out
        )

    gathers = {0: start_gather(0, 0)}
    scatters = {}
    for k in range(NTASKS):
        p = k & 1
        gathers[k].wait()
        if k >= 1:
            scatters[k - 1].wait()
        if k + 1 < NTASKS:
            gathers[k + 1] = start_gather(k + 1, 1 - p)
        scatters[k] = start_scatter(k, p)
    scatters[NTASKS - 1].wait()


def kernel(inputs):
    perm = jax.random.permutation(jax.random.key(0), BATCH).astype(jnp.int32)
    idxmat = (
        jnp.zeros((NW, 16), jnp.int32)
        .at[:, :ROWS_PER_W]
        .set(perm[:KEEP].reshape(NW, ROWS_PER_W))
    )
    x2d = inputs.reshape(BATCH, D)
    out2d, idx = _sc_gather(x2d, idxmat, perm)
    return out2d.reshape(KEEP, 512, 512), idx


# speedup vs baseline: 1.0007x; 1.0007x over previous
import functools

import jax
import jax.numpy as jnp
import numpy as np
from jax import lax
from jax.experimental import pallas as pl
from jax.experimental.pallas import tpu as pltpu
from jax.experimental.pallas import tpu_sc as plsc

BATCH = 128
KEEP = 96
D = 512 * 512

NC, NS = 2, 16
NW = NC * NS
ROWS_PER_W = KEEP // NW
CHUNK = 32768
NCHUNKS = D // CHUNK
NTASKS = ROWS_PER_W * NCHUNKS


@functools.partial(
    pl.kernel,
    out_type=(
        jax.ShapeDtypeStruct((KEEP, D), jnp.float32),
        jax.ShapeDtypeStruct((BATCH,), jnp.int32),
    ),
    mesh=plsc.VectorSubcoreMesh(core_axis_name="c", subcore_axis_name="s"),
    scratch_types=[
        pltpu.VMEM((16,), jnp.int32),
        pltpu.VMEM((BATCH,), jnp.int32),
        pltpu.VMEM((CHUNK,), jnp.float32),
        pltpu.VMEM((CHUNK,), jnp.float32),
        pltpu.VMEM((CHUNK,), jnp.float32),
        pltpu.SemaphoreType.DMA,
        pltpu.SemaphoreType.DMA,
        pltpu.SemaphoreType.DMA,
        pltpu.SemaphoreType.DMA,
        pltpu.SemaphoreType.DMA,
        pltpu.SemaphoreType.DMA,
    ],
)
def _sc_gather(x_hbm, idxmat_hbm, perm_hbm, out_hbm, idx_out_hbm,
               idx_v, perm_v, buf0, buf1, buf2,
               gsem0, gsem1, gsem2, ssem0, ssem1, ssem2):
    wid = lax.axis_index("s") * NC + lax.axis_index("c")

    @pl.when(wid == 0)
    def _():
        pltpu.sync_copy(perm_hbm, perm_v)
        pltpu.sync_copy(perm_v, idx_out_hbm)

    pltpu.sync_copy(idxmat_hbm.at[wid], idx_v)
    vec = idx_v[...]
    srcs = [vec[j] for j in range(ROWS_PER_W)]
    obase = wid * ROWS_PER_W

    bufs = (buf0, buf1, buf2)
    gsems = (gsem0, gsem1, gsem2)
    ssems = (ssem0, ssem1, ssem2)
    NBUF = 3
    tasks = [(j, c) for j in range(ROWS_PER_W) for c in range(NCHUNKS)]

    def start_gather(k):
        j, c = tasks[k]
        p = k % NBUF
        return pltpu.async_copy(
            x_hbm.at[srcs[j], pl.ds(c * CHUNK, CHUNK)], bufs[p], gsems[p]
        )

    def start_scatter(k):
        j, c = tasks[k]
        p = k % NBUF
        return pltpu.async_copy(
            bufs[p], out_hbm.at[obase + j, pl.ds(c * CHUNK, CHUNK)], ssems[p]
        )

    gathers = {k: start_gather(k) for k in range(2)}
    scatters = {}
    for k in range(NTASKS):
        gathers[k].wait()
        scatters[k] = start_scatter(k)
        if k + 2 < NTASKS:
            if k - 1 >= 0:
                scatters[k - 1].wait()
            gathers[k + 2] = start_gather(k + 2)
    scatters[NTASKS - 2].wait()
    scatters[NTASKS - 1].wait()


def kernel(inputs):
    perm = jax.random.permutation(jax.random.key(0), BATCH).astype(jnp.int32)
    idxmat = (
        jnp.zeros((NW, 16), jnp.int32)
        .at[:, :ROWS_PER_W]
        .set(perm[:KEEP].reshape(NW, ROWS_PER_W))
    )
    x2d = inputs.reshape(BATCH, D)
    out2d, idx = _sc_gather(x2d, idxmat, perm)
    return out2d.reshape(KEEP, 512, 512), idx


# speedup vs baseline: 1.0158x; 1.0151x over previous
import functools

import jax
import jax.numpy as jnp
import numpy as np
from jax import lax
from jax.experimental import pallas as pl
from jax.experimental.pallas import tpu as pltpu
from jax.experimental.pallas import tpu_sc as plsc

BATCH = 128
KEEP = 96
D = 512 * 512

NC, NS = 2, 16
NW = NC * NS
ROWS_PER_W = KEEP // NW
CHUNK = 32768
NCHUNKS = D // CHUNK
NTASKS = ROWS_PER_W * NCHUNKS


@functools.partial(
    pl.kernel,
    out_type=(
        jax.ShapeDtypeStruct((KEEP, D), jnp.float32),
        jax.ShapeDtypeStruct((BATCH,), jnp.int32),
    ),
    mesh=plsc.VectorSubcoreMesh(core_axis_name="c", subcore_axis_name="s"),
    scratch_types=[
        pltpu.VMEM((16,), jnp.int32),
        pltpu.VMEM((BATCH,), jnp.int32),
        pltpu.VMEM_SHARED((NS * 3 * CHUNK,), jnp.float32),
        pltpu.SemaphoreType.DMA,
        pltpu.SemaphoreType.DMA,
        pltpu.SemaphoreType.DMA,
        pltpu.SemaphoreType.DMA,
        pltpu.SemaphoreType.DMA,
        pltpu.SemaphoreType.DMA,
    ],
)
def _sc_gather(x_hbm, idxmat_hbm, perm_hbm, out_hbm, idx_out_hbm,
               idx_v, perm_v, ring,
               gsem0, gsem1, gsem2, ssem0, ssem1, ssem2):
    wid = lax.axis_index("s") * NC + lax.axis_index("c")

    @pl.when(wid == 0)
    def _():
        pltpu.sync_copy(perm_hbm, perm_v)
        pltpu.sync_copy(perm_v, idx_out_hbm)

    pltpu.sync_copy(idxmat_hbm.at[wid], idx_v)
    vec = idx_v[...]
    srcs = [vec[j] for j in range(ROWS_PER_W)]
    obase = wid * ROWS_PER_W

    sid = lax.axis_index("s")
    bufs = tuple(
        ring.at[pl.ds((sid * 3 + p) * CHUNK, CHUNK)] for p in range(3)
    )
    gsems = (gsem0, gsem1, gsem2)
    ssems = (ssem0, ssem1, ssem2)
    NBUF = 3
    tasks = [(j, c) for j in range(ROWS_PER_W) for c in range(NCHUNKS)]

    def start_gather(k):
        j, c = tasks[k]
        p = k % NBUF
        return pltpu.async_copy(
            x_hbm.at[srcs[j], pl.ds(c * CHUNK, CHUNK)], bufs[p], gsems[p]
        )

    def start_scatter(k):
        j, c = tasks[k]
        p = k % NBUF
        return pltpu.async_copy(
            bufs[p], out_hbm.at[obase + j, pl.ds(c * CHUNK, CHUNK)], ssems[p]
        )

    gathers = {k: start_gather(k) for k in range(2)}
    scatters = {}
    for k in range(NTASKS):
        gathers[k].wait()
        scatters[k] = start_scatter(k)
        if k + 2 < NTASKS:
            if k - 1 >= 0:
                scatters[k - 1].wait()
            gathers[k + 2] = start_gather(k + 2)
    scatters[NTASKS - 2].wait()
    scatters[NTASKS - 1].wait()


def kernel(inputs):
    perm = jax.random.permutation(jax.random.key(0), BATCH).astype(jnp.int32)
    idxmat = (
        jnp.zeros((NW, 16), jnp.int32)
        .at[:, :ROWS_PER_W]
        .set(perm[:KEEP].reshape(NW, ROWS_PER_W))
    )
    x2d = inputs.reshape(BATCH, D)
    out2d, idx = _sc_gather(x2d, idxmat, perm)
    return out2d.reshape(KEEP, 512, 512), idx
